# baseline (speedup 1.0000x reference)
import functools

import jax
import jax.numpy as jnp
from jax import lax
from jax.experimental import pallas as pl
from jax.experimental.pallas import tpu as pltpu
from jax.experimental.pallas import tpu_sc as plsc

E = 16
G = 4
EPG = E // G
K = 2
BLK = 1024

NC = 2
NS = 16
NW = NC * NS
L = 16


def _scores_kernel(w_ref, x_ref, s_ref):
    s_ref[...] = jax.lax.dot_general(
        w_ref[...], x_ref[...], (((1,), (1,)), ((), ())),
        preferred_element_type=jnp.float32)


def _route_kernel(tpw, s_hbm, wout_hbm, iout_hbm, s_v, wo_v, io_v):
    wid = lax.axis_index("s") * NC + lax.axis_index("c")
    base = wid * tpw
    pltpu.sync_copy(s_hbm.at[:, pl.ds(base, tpw)], s_v)

    neg = jnp.full((L,), -jnp.inf, dtype=jnp.float32)
    lane = lax.iota(jnp.int32, L)

    def body(t, carry):
        off = pl.multiple_of(t * L, L)
        vs = [s_v[e, pl.ds(off, L)] for e in range(E)]
        m = vs[0]
        for e in range(1, E):
            m = jnp.maximum(m, vs[e])
        ex = [jnp.exp(v - m) for v in vs]
        s = ex[0]
        for e in range(1, E):
            s = s + ex[e]
        p = [v / s for v in ex]
        gs = [p[g * EPG] for g in range(G)]
        for g in range(G):
            for j in range(1, EPG):
                gs[g] = jnp.maximum(gs[g], p[g * EPG + j])
        gconst = [jnp.full((L,), g, dtype=jnp.int32) for g in range(G + 1)]
        g1v = jnp.maximum(jnp.maximum(gs[0], gs[1]),
                          jnp.maximum(gs[2], gs[3]))
        g1 = gconst[G]
        for g in range(G - 1, -1, -1):
            g1 = jnp.where(gs[g] == g1v, gconst[g], g1)
        gs2 = [jnp.where(g1 == gconst[g], neg, gs[g]) for g in range(G)]
        g2v = jnp.maximum(jnp.maximum(gs2[0], gs2[1]),
                          jnp.maximum(gs2[2], gs2[3]))
        g2 = gconst[G]
        for g in range(G - 1, -1, -1):
            g2 = jnp.where(gs2[g] == g2v, gconst[g], g2)
        sel = []
        for e in range(E):
            ge = gconst[e // EPG]
            allowed = (g1 == ge) | (g2 == ge)
            sel.append(jnp.where(allowed, p[e], neg))
        econst = [jnp.full((L,), e, dtype=jnp.int32) for e in range(E + 1)]
        m1 = sel[0]
        for e in range(1, E):
            m1 = jnp.maximum(m1, sel[e])
        i1 = econst[E]
        for e in range(E - 1, -1, -1):
            i1 = jnp.where(sel[e] == m1, econst[e], i1)
        sel2 = [jnp.where(i1 == econst[e], neg, sel[e]) for e in range(E)]
        m2 = sel2[0]
        for e in range(1, E):
            m2 = jnp.maximum(m2, sel2[e])
        i2 = econst[E]
        for e in range(E - 1, -1, -1):
            i2 = jnp.where(sel2[e] == m2, econst[e], i2)
        pos = (off + lane) * K
        plsc.store_scatter(wo_v, [pos], m1)
        plsc.store_scatter(wo_v, [pos + 1], m2)
        plsc.store_scatter(io_v, [pos], i1)
        plsc.store_scatter(io_v, [pos + 1], i2)
        return carry

    lax.fori_loop(0, tpw // L, body, 0)
    pltpu.sync_copy(wo_v, wout_hbm.at[pl.ds(base * K, tpw * K)])
    pltpu.sync_copy(io_v, iout_hbm.at[pl.ds(base * K, tpw * K)])


@jax.jit
def kernel(x, W):
    n, d = x.shape
    scores_t = pl.pallas_call(
        _scores_kernel,
        grid=(n // BLK,),
        in_specs=[pl.BlockSpec((E, d), lambda i: (0, 0)),
                  pl.BlockSpec((BLK, d), lambda i: (i, 0))],
        out_specs=pl.BlockSpec((E, BLK), lambda i: (0, i)),
        out_shape=jax.ShapeDtypeStruct((E, n), jnp.float32),
    )(W, x)

    tpw = n // NW
    route = pl.kernel(
        functools.partial(_route_kernel, tpw),
        out_type=[jax.ShapeDtypeStruct((n * K,), jnp.float32),
                  jax.ShapeDtypeStruct((n * K,), jnp.int32)],
        mesh=plsc.VectorSubcoreMesh(core_axis_name="c", subcore_axis_name="s"),
        scratch_types=[pltpu.VMEM((E, tpw), jnp.float32),
                       pltpu.VMEM((tpw * K,), jnp.float32),
                       pltpu.VMEM((tpw * K,), jnp.int32)],
        compiler_params=pltpu.CompilerParams(needs_layout_passes=False),
    )
    wout, iout = route(scores_t)
    return wout.reshape(n, K), iout.reshape(n, K)


# speedup vs baseline: 1.5454x; 1.5454x over previous
import functools

import jax
import jax.numpy as jnp
from jax import lax
from jax.experimental import pallas as pl
from jax.experimental.pallas import tpu as pltpu
from jax.experimental.pallas import tpu_sc as plsc

E = 16
G = 4
EPG = E // G
K = 2
BLK = 1024

NC = 2
NS = 16
NW = NC * NS
L = 16


def _scores_kernel(w_ref, x_ref, s_ref):
    s_ref[...] = jax.lax.dot_general(
        w_ref[...], x_ref[...], (((1,), (1,)), ((), ())),
        preferred_element_type=jnp.float32)


def _route_kernel(tpw, s_hbm, wout_hbm, iout_hbm, s_v, wo_v, io_v):
    wid = lax.axis_index("s") * NC + lax.axis_index("c")
    base = wid * tpw
    pltpu.sync_copy(s_hbm.at[:, pl.ds(base, tpw)], s_v)

    neg = jnp.full((L,), -jnp.inf, dtype=jnp.float32)

    def maxtree(xs):
        xs = list(xs)
        while len(xs) > 1:
            xs = [jnp.maximum(xs[i], xs[i + 1])
                  for i in range(0, len(xs) - 1, 2)] + (
                      [xs[-1]] if len(xs) % 2 else [])
        return xs[0]

    def sumtree(xs):
        xs = list(xs)
        while len(xs) > 1:
            xs = [xs[i] + xs[i + 1]
                  for i in range(0, len(xs) - 1, 2)] + (
                      [xs[-1]] if len(xs) % 2 else [])
        return xs[0]

    def argmaxtree(vals, idxs):
        vs, ids = list(vals), list(idxs)
        while len(vs) > 1:
            nv, ni = [], []
            for i in range(0, len(vs) - 1, 2):
                take = vs[i] >= vs[i + 1]
                nv.append(jnp.where(take, vs[i], vs[i + 1]))
                ni.append(jnp.where(take, ids[i], ids[i + 1]))
            if len(vs) % 2:
                nv.append(vs[-1])
                ni.append(ids[-1])
            vs, ids = nv, ni
        return vs[0], ids[0]

    def body(t, carry):
        off = pl.multiple_of(t * L, L)
        vs = [s_v[e, pl.ds(off, L)] for e in range(E)]
        m = maxtree(vs)
        ex = [jnp.exp(v - m) for v in vs]
        s = sumtree(ex)
        p = [v / s for v in ex]
        gs = [maxtree(p[g * EPG:(g + 1) * EPG]) for g in range(G)]
        gconst = [jnp.full((L,), g, dtype=jnp.int32) for g in range(G)]
        _, g1 = argmaxtree(gs, gconst)
        gs2 = [jnp.where(g1 == gconst[g], neg, gs[g]) for g in range(G)]
        _, g2 = argmaxtree(gs2, gconst)
        sel = []
        for e in range(E):
            ge = gconst[e // EPG]
            allowed = (g1 == ge) | (g2 == ge)
            sel.append(jnp.where(allowed, p[e], neg))
        econst = [jnp.full((L,), e, dtype=jnp.int32) for e in range(E)]
        m1, i1 = argmaxtree(sel, econst)
        sel2 = [jnp.where(i1 == econst[e], neg, sel[e]) for e in range(E)]
        m2, i2 = argmaxtree(sel2, econst)
        wo_v[0, pl.ds(off, L)] = m1
        wo_v[1, pl.ds(off, L)] = m2
        io_v[0, pl.ds(off, L)] = i1
        io_v[1, pl.ds(off, L)] = i2
        return carry

    lax.fori_loop(0, tpw // L, body, 0)
    pltpu.sync_copy(wo_v, wout_hbm.at[:, pl.ds(base, tpw)])
    pltpu.sync_copy(io_v, iout_hbm.at[:, pl.ds(base, tpw)])


@jax.jit
def kernel(x, W):
    n, d = x.shape
    scores_t = pl.pallas_call(
        _scores_kernel,
        grid=(n // BLK,),
        in_specs=[pl.BlockSpec((E, d), lambda i: (0, 0)),
                  pl.BlockSpec((BLK, d), lambda i: (i, 0))],
        out_specs=pl.BlockSpec((E, BLK), lambda i: (0, i)),
        out_shape=jax.ShapeDtypeStruct((E, n), jnp.float32),
    )(W, x)

    tpw = n // NW
    route = pl.kernel(
        functools.partial(_route_kernel, tpw),
        out_type=[jax.ShapeDtypeStruct((K, n), jnp.float32),
                  jax.ShapeDtypeStruct((K, n), jnp.int32)],
        mesh=plsc.VectorSubcoreMesh(core_axis_name="c", subcore_axis_name="s"),
        scratch_types=[pltpu.VMEM((E, tpw), jnp.float32),
                       pltpu.VMEM((K, tpw), jnp.float32),
                       pltpu.VMEM((K, tpw), jnp.int32)],
    )
    wout, iout = route(scores_t)
    return wout.T, iout.T
